# baseline (device time: 113106 ns/iter reference)
import jax
import jax.numpy as jnp
from jax import lax
from jax.experimental import pallas as pl
from jax.experimental.pallas import tpu as pltpu

N_DEV = 16
N_TOK = 1024
N_EXP = 64
CAP = 12
EXP_PER_DEV = N_EXP // N_DEV
ROWS_PER_DEV = N_TOK // N_DEV


def _ring_reduce_scatter(c):
    n, h = c.shape
    m = ROWS_PER_DEV

    def body(c_ref, out_ref, sbuf, rbuf, send_sems, recv_sems):
        me = lax.axis_index("i")
        left = (me + N_DEV - 1) % N_DEV
        right = (me + 1) % N_DEV

        barrier_sem = pltpu.get_barrier_semaphore()
        for nbr in (left, right):
            pl.semaphore_signal(
                barrier_sem, inc=1,
                device_id=(nbr,), device_id_type=pl.DeviceIdType.MESH,
            )
        pl.semaphore_wait(barrier_sem, 2)

        seed_chunk = (me + N_DEV - 1) % N_DEV
        sbuf[:, :] = c_ref[pl.ds(seed_chunk * m, m), :]

        for hop in range(N_DEV - 1):
            src = sbuf if hop == 0 else rbuf.at[hop - 1]
            rdma = pltpu.make_async_remote_copy(
                src_ref=src,
                dst_ref=rbuf.at[hop],
                send_sem=send_sems.at[hop],
                recv_sem=recv_sems.at[hop],
                device_id=(right,),
                device_id_type=pl.DeviceIdType.MESH,
            )
            rdma.start()
            rdma.wait()

            add_chunk = (me + 2 * N_DEV - 2 - hop) % N_DEV
            contrib = c_ref[pl.ds(add_chunk * m, m), :]
            if hop < N_DEV - 2:
                rbuf[hop, :, :] = rbuf[hop, :, :] + contrib
            else:
                out_ref[:, :] = rbuf[hop, :, :] + contrib

    return pl.pallas_call(
        body,
        out_shape=jax.ShapeDtypeStruct((m, h), c.dtype),
        in_specs=[pl.BlockSpec(memory_space=pltpu.VMEM)],
        out_specs=pl.BlockSpec(memory_space=pltpu.VMEM),
        scratch_shapes=[
            pltpu.VMEM((m, h), c.dtype),
            pltpu.VMEM((N_DEV - 1, m, h), c.dtype),
            pltpu.SemaphoreType.DMA((N_DEV - 1,)),
            pltpu.SemaphoreType.DMA((N_DEV - 1,)),
        ],
        compiler_params=pltpu.CompilerParams(collective_id=0),
    )(c)


def kernel(x, router_W, route_idx, expert_W):
    del router_W
    n, d = x.shape
    e_local, _, h = expert_W.shape
    my = lax.axis_index("i")

    e = route_idx[:, 0].astype(jnp.int32)
    onehot = (e[:, None] == jnp.arange(N_EXP, dtype=jnp.int32)[None, :])
    cum = jnp.cumsum(onehot.astype(jnp.int32), axis=0)
    p = jnp.take_along_axis(cum, e[:, None], axis=1)[:, 0] - 1

    tokslot = jnp.full((N_EXP, CAP), n, jnp.int32)
    tokslot = tokslot.at[e, p].set(jnp.arange(n, dtype=jnp.int32), mode="drop")

    my_tok = lax.dynamic_slice(tokslot, (my * EXP_PER_DEV, 0), (EXP_PER_DEV, CAP))
    valid = my_tok < n
    xg = x[jnp.where(valid, my_tok, 0)] * valid[..., None]
    y = jnp.einsum("kcd,kdh->kch", xg, expert_W)

    c = jnp.zeros((n, h), jnp.float32)
    c = c.at[my_tok.reshape(-1)].set(y.reshape(-1, h), mode="drop")

    return _ring_reduce_scatter(c)


# device time: 69318 ns/iter; 1.6317x vs baseline; 1.6317x over previous
import functools

import jax
import jax.numpy as jnp
from jax import lax
from jax.experimental import pallas as pl
from jax.experimental.pallas import tpu as pltpu

N_DEV = 16
N_TOK = 1024
N_EXP = 64
CAP = 12
EXP_PER_DEV = N_EXP // N_DEV
ROWS_PER_DEV = N_TOK // N_DEV
ZERO_ROW = EXP_PER_DEV * CAP


def _row_scatter(ybuf, meta, nm, h):

    def body(ybuf_ref, meta_ref, nm_ref, out_ref, send_sem, recv_sem):
        me = lax.axis_index("i")

        barrier_sem = pltpu.get_barrier_semaphore()
        for k in range(1, N_DEV):
            pl.semaphore_signal(
                barrier_sem, inc=1,
                device_id=((me + k) % N_DEV,),
                device_id_type=pl.DeviceIdType.MESH,
            )
        pl.semaphore_wait(barrier_sem, N_DEV - 1)

        n_mine = nm_ref[0]

        def send_one(j, carry):
            s = meta_ref[0, j]
            ddev = meta_ref[1, j]
            drow = meta_ref[2, j]
            rdma = pltpu.make_async_remote_copy(
                src_ref=ybuf_ref.at[pl.ds(s, 1)],
                dst_ref=out_ref.at[pl.ds(drow, 1)],
                send_sem=send_sem,
                recv_sem=recv_sem,
                device_id=(ddev,),
                device_id_type=pl.DeviceIdType.MESH,
            )
            rdma.start()
            return carry

        lax.fori_loop(0, n_mine, send_one, 0)

        dummy = pltpu.make_async_remote_copy(
            src_ref=ybuf_ref.at[pl.ds(ZERO_ROW, 1)],
            dst_ref=out_ref.at[pl.ds(0, 1)],
            send_sem=send_sem,
            recv_sem=recv_sem,
            device_id=(me,),
            device_id_type=pl.DeviceIdType.MESH,
        )
        for _ in range(ROWS_PER_DEV):
            dummy.wait_recv()
        lax.fori_loop(0, n_mine, lambda j, c: (dummy.wait_send(), c)[1], 0)

        @functools.partial(
            pl.run_scoped, second_barrier=pltpu.SemaphoreType.REGULAR
        )
        def _(second_barrier):
            for k in range(1, N_DEV):
                pl.semaphore_signal(
                    second_barrier, inc=1,
                    device_id=((me + k) % N_DEV,),
                    device_id_type=pl.DeviceIdType.MESH,
                )
            pl.semaphore_wait(second_barrier, N_DEV - 1)

    return pl.pallas_call(
        body,
        out_shape=jax.ShapeDtypeStruct((ROWS_PER_DEV, h), ybuf.dtype),
        in_specs=[
            pl.BlockSpec(memory_space=pltpu.VMEM),
            pl.BlockSpec(memory_space=pltpu.SMEM),
            pl.BlockSpec(memory_space=pltpu.SMEM),
        ],
        out_specs=pl.BlockSpec(memory_space=pltpu.VMEM),
        scratch_shapes=[
            pltpu.SemaphoreType.DMA,
            pltpu.SemaphoreType.DMA,
        ],
        compiler_params=pltpu.CompilerParams(collective_id=0),
    )(ybuf, meta, nm)


def kernel(x, router_W, route_idx, expert_W):
    del router_W
    n, d = x.shape
    e_local, _, h = expert_W.shape
    my = lax.axis_index("i")

    e = route_idx[:, 0].astype(jnp.int32)
    onehot = (e[:, None] == jnp.arange(N_EXP, dtype=jnp.int32)[None, :])
    cum = jnp.cumsum(onehot.astype(jnp.int32), axis=0)
    p = jnp.take_along_axis(cum, e[:, None], axis=1)[:, 0] - 1
    keep = p < CAP

    tokslot = jnp.full((N_EXP, CAP), n, jnp.int32)
    tokslot = tokslot.at[e, p].set(jnp.arange(n, dtype=jnp.int32), mode="drop")

    my_tok = lax.dynamic_slice(tokslot, (my * EXP_PER_DEV, 0), (EXP_PER_DEV, CAP))
    valid = my_tok < n
    xg = x[jnp.where(valid, my_tok, 0)] * valid[..., None]
    y = jnp.einsum("kcd,kdh->kch", xg, expert_W)
    ybuf = jnp.concatenate(
        [y.reshape(EXP_PER_DEV * CAP, h), jnp.zeros((1, h), jnp.float32)]
    )

    owner = e // EXP_PER_DEV
    srcrow = jnp.where(keep, (e % EXP_PER_DEV) * CAP + p, ZERO_ROW)
    mine = owner == my
    order = jnp.argsort(jnp.where(mine, 0, 1), stable=True)
    n_mine = jnp.sum(mine.astype(jnp.int32))
    meta = jnp.stack(
        [srcrow[order], order // ROWS_PER_DEV, order % ROWS_PER_DEV]
    ).astype(jnp.int32)
    nm = n_mine.reshape(1).astype(jnp.int32)

    return _row_scatter(ybuf, meta, nm, h)


# device time: 22375 ns/iter; 5.0550x vs baseline; 3.0980x over previous
import functools

import jax
import jax.numpy as jnp
from jax import lax
from jax.experimental import pallas as pl
from jax.experimental.pallas import tpu as pltpu

N_DEV = 16
N_TOK = 1024
N_EXP = 64
CAP = 12
EXP_PER_DEV = N_EXP // N_DEV
N_SLOT = EXP_PER_DEV * CAP
ROWS_PER_DEV = N_TOK // N_DEV


def _row_scatter(ybuf, meta, nm, h):

    def body(ybuf_ref, meta_ref, nm_ref, out_ref, send_sem, recv_sem):
        me = lax.axis_index("i")

        out_ref[:, :] = jnp.zeros((ROWS_PER_DEV, h), jnp.float32)

        barrier_sem = pltpu.get_barrier_semaphore()
        for k in range(1, N_DEV):
            pl.semaphore_signal(
                barrier_sem, inc=1,
                device_id=((me + k) % N_DEV,),
                device_id_type=pl.DeviceIdType.MESH,
            )
        pl.semaphore_wait(barrier_sem, N_DEV - 1)

        for j in range(N_SLOT):
            valid = meta_ref[0, j]
            ddev = meta_ref[1, j]
            drow = meta_ref[2, j]

            @pl.when(valid == 1)
            def _():
                rdma = pltpu.make_async_remote_copy(
                    src_ref=ybuf_ref.at[pl.ds(j, 1)],
                    dst_ref=out_ref.at[pl.ds(drow, 1)],
                    send_sem=send_sem,
                    recv_sem=recv_sem,
                    device_id=(ddev,),
                    device_id_type=pl.DeviceIdType.MESH,
                )
                rdma.start()

        dummy = pltpu.make_async_remote_copy(
            src_ref=ybuf_ref.at[pl.ds(0, 1)],
            dst_ref=out_ref.at[pl.ds(0, 1)],
            send_sem=send_sem,
            recv_sem=recv_sem,
            device_id=(me,),
            device_id_type=pl.DeviceIdType.MESH,
        )
        lax.fori_loop(0, nm_ref[1], lambda j, c: (dummy.wait_recv(), c)[1], 0)
        lax.fori_loop(0, nm_ref[0], lambda j, c: (dummy.wait_send(), c)[1], 0)

        @functools.partial(
            pl.run_scoped, second_barrier=pltpu.SemaphoreType.REGULAR
        )
        def _(second_barrier):
            for k in range(1, N_DEV):
                pl.semaphore_signal(
                    second_barrier, inc=1,
                    device_id=((me + k) % N_DEV,),
                    device_id_type=pl.DeviceIdType.MESH,
                )
            pl.semaphore_wait(second_barrier, N_DEV - 1)

    return pl.pallas_call(
        body,
        out_shape=jax.ShapeDtypeStruct((ROWS_PER_DEV, h), ybuf.dtype),
        in_specs=[
            pl.BlockSpec(memory_space=pltpu.VMEM),
            pl.BlockSpec(memory_space=pltpu.SMEM),
            pl.BlockSpec(memory_space=pltpu.SMEM),
        ],
        out_specs=pl.BlockSpec(memory_space=pltpu.VMEM),
        scratch_shapes=[
            pltpu.SemaphoreType.DMA,
            pltpu.SemaphoreType.DMA,
        ],
        compiler_params=pltpu.CompilerParams(collective_id=0),
    )(ybuf, meta, nm)


def kernel(x, router_W, route_idx, expert_W):
    del router_W
    n, d = x.shape
    e_local, _, h = expert_W.shape
    my = lax.axis_index("i")

    e = route_idx[:, 0].astype(jnp.int32)
    onehot = (e[:, None] == jnp.arange(N_EXP, dtype=jnp.int32)[None, :]).astype(
        jnp.float32
    )
    tril = jnp.tril(jnp.ones((n, n), jnp.float32))
    cum = jnp.dot(tril, onehot, precision=lax.Precision.HIGHEST)
    p = jnp.rint(jnp.sum(onehot * cum, axis=1) - 1.0).astype(jnp.int32)
    keep = p < CAP

    peq = (
        (p[:, None] == jnp.arange(CAP, dtype=jnp.int32)[None, :])
        & keep[:, None]
    ).astype(jnp.float32)
    tokp1 = jnp.rint(
        jnp.einsum(
            "tE,tc->Ec",
            onehot,
            peq * jnp.arange(1, n + 1, dtype=jnp.float32)[:, None],
            precision=lax.Precision.HIGHEST,
        )
    )

    myslot = lax.dynamic_slice(
        tokp1, (my * EXP_PER_DEV, 0), (EXP_PER_DEV, CAP)
    ).reshape(N_SLOT).astype(jnp.int32)
    valid = myslot > 0
    tok = jnp.maximum(myslot - 1, 0)

    sel = (
        (tok[:, None] == jnp.arange(n, dtype=jnp.int32)[None, :])
        & valid[:, None]
    ).astype(jnp.float32)
    xg = (sel @ x).reshape(EXP_PER_DEV, CAP, d)
    ybuf = jnp.einsum("kcd,kdh->kch", xg, expert_W).reshape(N_SLOT, h)

    meta = jnp.stack(
        [valid.astype(jnp.int32), tok // ROWS_PER_DEV, tok % ROWS_PER_DEV]
    )
    n_sent = jnp.sum(valid.astype(jnp.int32))
    myblock_keep = lax.dynamic_slice(
        keep.astype(jnp.int32), (my * ROWS_PER_DEV,), (ROWS_PER_DEV,)
    )
    n_recv = jnp.sum(myblock_keep)
    nm = jnp.stack([n_sent, n_recv]).astype(jnp.int32)

    return _row_scatter(ybuf, meta, nm, h)


# device time: 21189 ns/iter; 5.3380x vs baseline; 1.0560x over previous
import functools

import jax
import jax.numpy as jnp
from jax import lax
from jax.experimental import pallas as pl
from jax.experimental.pallas import tpu as pltpu

N_DEV = 16
N_TOK = 1024
N_EXP = 64
CAP = 12
EXP_PER_DEV = N_EXP // N_DEV
N_SLOT = EXP_PER_DEV * CAP
ROWS_PER_DEV = N_TOK // N_DEV


def _row_scatter(ybuf, meta, nm, h):

    def body(ybuf_ref, meta_ref, nm_ref, out_ref, send_sem, recv_sem):
        me = lax.axis_index("i")

        out_ref[:, :] = jnp.zeros((ROWS_PER_DEV, h), jnp.float32)

        barrier_sem = pltpu.get_barrier_semaphore()
        for k in range(1, N_DEV):
            pl.semaphore_signal(
                barrier_sem, inc=1,
                device_id=((me + k) % N_DEV,),
                device_id_type=pl.DeviceIdType.MESH,
            )
        pl.semaphore_wait(barrier_sem, N_DEV - 1)

        for j in range(N_SLOT):
            valid = meta_ref[0, j]
            ddev = meta_ref[1, j]
            drow = meta_ref[2, j]

            @pl.when(valid == 1)
            def _():
                rdma = pltpu.make_async_remote_copy(
                    src_ref=ybuf_ref.at[pl.ds(j, 1)],
                    dst_ref=out_ref.at[pl.ds(drow, 1)],
                    send_sem=send_sem,
                    recv_sem=recv_sem,
                    device_id=(ddev,),
                    device_id_type=pl.DeviceIdType.MESH,
                )
                rdma.start()

        dummy = pltpu.make_async_remote_copy(
            src_ref=ybuf_ref.at[pl.ds(0, 1)],
            dst_ref=out_ref.at[pl.ds(0, 1)],
            send_sem=send_sem,
            recv_sem=recv_sem,
            device_id=(me,),
            device_id_type=pl.DeviceIdType.MESH,
        )
        lax.fori_loop(0, nm_ref[1], lambda j, c: (dummy.wait_recv(), c)[1], 0)
        lax.fori_loop(0, nm_ref[0], lambda j, c: (dummy.wait_send(), c)[1], 0)

        @functools.partial(
            pl.run_scoped, second_barrier=pltpu.SemaphoreType.REGULAR
        )
        def _(second_barrier):
            for k in range(1, N_DEV):
                pl.semaphore_signal(
                    second_barrier, inc=1,
                    device_id=((me + k) % N_DEV,),
                    device_id_type=pl.DeviceIdType.MESH,
                )
            pl.semaphore_wait(second_barrier, N_DEV - 1)

    return pl.pallas_call(
        body,
        out_shape=jax.ShapeDtypeStruct((ROWS_PER_DEV, h), ybuf.dtype),
        in_specs=[
            pl.BlockSpec(memory_space=pltpu.VMEM),
            pl.BlockSpec(memory_space=pltpu.SMEM),
            pl.BlockSpec(memory_space=pltpu.SMEM),
        ],
        out_specs=pl.BlockSpec(memory_space=pltpu.VMEM),
        scratch_shapes=[
            pltpu.SemaphoreType.DMA,
            pltpu.SemaphoreType.DMA,
        ],
        compiler_params=pltpu.CompilerParams(collective_id=0),
    )(ybuf, meta, nm)


def kernel(x, router_W, route_idx, expert_W):
    del router_W
    n, d = x.shape
    e_local, _, h = expert_W.shape
    my = lax.axis_index("i")

    e = route_idx[:, 0].astype(jnp.int32)
    onehot = (e[:, None] == jnp.arange(N_EXP, dtype=jnp.int32)[None, :]).astype(
        jnp.float32
    )
    tril = jnp.tril(jnp.ones((n, n), jnp.float32))
    cum = jnp.dot(tril, onehot)
    p = jnp.rint(jnp.sum(onehot * cum, axis=1) - 1.0).astype(jnp.int32)
    keep = p < CAP

    peq = (
        (p[:, None] == jnp.arange(CAP, dtype=jnp.int32)[None, :])
        & keep[:, None]
    ).astype(jnp.float32)
    tokp1 = jnp.rint(
        jnp.einsum(
            "tE,tc->Ec",
            onehot,
            peq * jnp.arange(1, n + 1, dtype=jnp.float32)[:, None],
            precision=lax.Precision.HIGHEST,
        )
    )

    myslot = lax.dynamic_slice(
        tokp1, (my * EXP_PER_DEV, 0), (EXP_PER_DEV, CAP)
    ).reshape(N_SLOT).astype(jnp.int32)
    valid = myslot > 0
    tok = jnp.maximum(myslot - 1, 0)

    sel = (
        (tok[:, None] == jnp.arange(n, dtype=jnp.int32)[None, :])
        & valid[:, None]
    ).astype(jnp.float32)
    xg = (sel @ x).reshape(EXP_PER_DEV, CAP, d)
    ybuf = jnp.einsum("kcd,kdh->kch", xg, expert_W).reshape(N_SLOT, h)

    meta = jnp.stack(
        [valid.astype(jnp.int32), tok // ROWS_PER_DEV, tok % ROWS_PER_DEV]
    )
    n_sent = jnp.sum(valid.astype(jnp.int32))
    myblock_keep = lax.dynamic_slice(
        keep.astype(jnp.int32), (my * ROWS_PER_DEV,), (ROWS_PER_DEV,)
    )
    n_recv = jnp.sum(myblock_keep)
    nm = jnp.stack([n_sent, n_recv]).astype(jnp.int32)

    return _row_scatter(ybuf, meta, nm, h)
